# baseline (speedup 1.0000x reference)
import functools

import jax
import jax.numpy as jnp
from jax import lax
from jax.experimental import pallas as pl
from jax.experimental.pallas import tpu as pltpu
from jax.experimental.pallas import tpu_sc as plsc

ACT = 2048
DICT = 16384
TOKENS = 4096
K = 64
K_TOTAL = K * TOKENS
L1_COEFF = 1e-4
NTOT = TOKENS * DICT


def _encode_body(x_ref, b_ref, w_ref, o_ref):
    xc = x_ref[...] - b_ref[...]
    o_ref[...] = lax.dot_general(
        xc, w_ref[...], (((1,), (0,)), ((), ())),
        preferred_element_type=jnp.float32)


def _encode(x, W_enc, b_dec):
    BM, BN = 2048, 512
    grid = (TOKENS // BM, DICT // BN)
    return pl.pallas_call(
        _encode_body,
        grid=grid,
        in_specs=[
            pl.BlockSpec((BM, ACT), lambda i, j: (i, 0)),
            pl.BlockSpec((1, ACT), lambda i, j: (0, 0)),
            pl.BlockSpec((ACT, BN), lambda i, j: (0, j)),
        ],
        out_specs=pl.BlockSpec((BM, BN), lambda i, j: (i, j)),
        out_shape=jax.ShapeDtypeStruct((TOKENS, DICT), jnp.float32),
    )(x, b_dec.reshape(1, ACT), W_enc)


def _decode_body(thr_ref, a_ref, w_ref, x_ref, b_ref,
                 topk_ref, sae_ref, stats_ref, acc_ref, sacc_ref,
                 *, n_m, n_k):
    m = pl.program_id(0)
    kk = pl.program_id(1)
    t = thr_ref[0, 0]

    a = a_ref[...]
    sel = jnp.abs(a) >= t
    av = jnp.where(sel, a, 0.0)
    topk_ref[...] = av

    @pl.when(jnp.logical_and(m == 0, kk == 0))
    def _init_stats():
        for i in range(5):
            sacc_ref[i] = 0.0

    sacc_ref[0] += jnp.sum(jnp.abs(av))
    sacc_ref[1] += jnp.sum(sel.astype(jnp.float32))
    sacc_ref[2] += jnp.sum((av > 0).astype(jnp.float32))
    sacc_ref[3] += jnp.sum((av < 0).astype(jnp.float32))

    @pl.when(kk == 0)
    def _zero_acc():
        acc_ref[...] = jnp.zeros_like(acc_ref)

    acc_ref[...] += lax.dot_general(
        av.astype(jnp.bfloat16), w_ref[...], (((1,), (0,)), ((), ())),
        preferred_element_type=jnp.float32)

    @pl.when(kk == n_k - 1)
    def _finish_row():
        rec = acc_ref[...] + b_ref[...]
        sae_ref[...] = rec
        sacc_ref[4] += jnp.sum((rec - x_ref[...]) ** 2)

    @pl.when(jnp.logical_and(m == n_m - 1, kk == n_k - 1))
    def _emit_stats():
        lane = lax.broadcasted_iota(jnp.int32, (1, 128), 1)
        out = jnp.zeros((1, 128), jnp.float32)
        for i in range(5):
            out = jnp.where(lane == i, sacc_ref[i], out)
        stats_ref[...] = out


def _decode(thr2d, acts, W_dec_bf16, x, b_dec):
    BM, BK = 1024, 256
    n_m, n_k = TOKENS // BM, DICT // BK
    body = functools.partial(_decode_body, n_m=n_m, n_k=n_k)
    return pl.pallas_call(
        body,
        grid=(n_m, n_k),
        in_specs=[
            pl.BlockSpec(memory_space=pltpu.SMEM),
            pl.BlockSpec((BM, BK), lambda m, k: (m, k)),
            pl.BlockSpec((BK, ACT), lambda m, k: (k, 0)),
            pl.BlockSpec((BM, ACT), lambda m, k: (m, 0)),
            pl.BlockSpec((1, ACT), lambda m, k: (0, 0)),
        ],
        out_specs=[
            pl.BlockSpec((BM, BK), lambda m, k: (m, k)),
            pl.BlockSpec((BM, ACT), lambda m, k: (m, 0)),
            pl.BlockSpec((1, 128), lambda m, k: (0, 0)),
        ],
        out_shape=[
            jax.ShapeDtypeStruct((TOKENS, DICT), jnp.float32),
            jax.ShapeDtypeStruct((TOKENS, ACT), jnp.float32),
            jax.ShapeDtypeStruct((1, 128), jnp.float32),
        ],
        scratch_shapes=[
            pltpu.VMEM((BM, ACT), jnp.float32),
            pltpu.SMEM((8,), jnp.float32),
        ],
    )(thr2d, acts, W_dec_bf16, x, b_dec.reshape(1, ACT))


_NW = 32
_SHARD = NTOT // _NW
_CHUNK = DICT
_NCHUNK = _SHARD // _CHUNK

_PA_BINS, _PA_PAD, _PA_SPLIT = 4096, 4096, 16
_PB_BINS, _PB_PAD, _PB_SPLIT = 2048, 2304, 3
_PC_BINS, _PC_PAD, _PC_SPLIT = 256, 512, 1


def _sc_mesh():
    return plsc.VectorSubcoreMesh(core_axis_name="c", subcore_axis_name="s")


def _worker_id():
    return lax.axis_index("s") * 2 + lax.axis_index("c")


def _zero_i32(ref, n):
    z = jnp.zeros((16,), jnp.int32)

    def bd(i, _):
        for u in range(4):
            ref[pl.ds(i * 64 + u * 16, 16)] = z
        return 0

    lax.fori_loop(0, n // 64, bd, 0)


def _hist_shard(acts_hbm, hist_ref, buf0, buf1, sem0, sem1, idx_fn):
    base = _worker_id() * _NCHUNK
    lane = lax.iota(jnp.int32, 16)
    ones = jnp.ones((16,), jnp.int32)

    def start(buf, sem, row):
        pltpu.async_copy(acts_hbm.at[row], buf, sem)

    def wait(buf, sem, row):
        pltpu.make_async_copy(acts_hbm.at[row], buf, sem).wait()

    def process(buf):
        @plsc.parallel_loop(0, _CHUNK // 16, unroll=8)
        def _body(i):
            v = buf[pl.ds(i * 16, 16)]
            bits = plsc.bitcast(v, jnp.int32)
            idx = idx_fn(bits, lane)
            plsc.addupdate_scatter(hist_ref, [idx], ones)

    start(buf0, sem0, base)

    def outer(i, _):
        row_a = base + 2 * i
        start(buf1, sem1, row_a + 1)
        wait(buf0, sem0, row_a)
        process(buf0)

        @pl.when(i < _NCHUNK // 2 - 1)
        def _():
            start(buf0, sem0, row_a + 2)

        wait(buf1, sem1, row_a + 1)
        process(buf1)
        return 0

    lax.fori_loop(0, _NCHUNK // 2, outer, 0)


def _reduce_fold(hist_ref, shared, red, tmp, fold, out_hbm, npad, nsplit):
    c = lax.axis_index("c")
    s = lax.axis_index("s")
    lane = lax.iota(jnp.int32, 16)
    nbw = npad * 16
    seg = nbw // nsplit
    sl = seg // 16
    bins_pt = sl // 16

    def split_body(split, _):
        pltpu.sync_copy(hist_ref.at[pl.ds(split * seg, seg)], shared.at[s])
        plsc.subcore_barrier()
        pltpu.sync_copy(shared.at[0, pl.ds(s * sl, sl)], red)

        def row_body(row, _):
            pltpu.sync_copy(shared.at[row, pl.ds(s * sl, sl)], tmp)

            def add_bd(i, _):
                red[pl.ds(i * 16, 16)] += tmp[pl.ds(i * 16, 16)]
                return 0

            lax.fori_loop(0, sl // 16, add_bd, 0)
            return 0

        lax.fori_loop(1, 16, row_body, 0)

        def fold_g(g, _):
            def fold_j(j, acc):
                vec = red[pl.ds((g * 16 + j) * 16, 16)]
                return jnp.where(lane == j, jnp.sum(vec), acc)

            acc = lax.fori_loop(0, 16, fold_j, jnp.zeros((16,), jnp.int32))
            fold[pl.ds(g * 16, 16)] = acc
            return 0

        lax.fori_loop(0, bins_pt // 16, fold_g, 0)
        pltpu.sync_copy(
            fold.at[pl.ds(0, bins_pt)],
            out_hbm.at[pl.ds(c * npad + split * (seg // 16) + s * bins_pt,
                             bins_pt)])
        plsc.subcore_barrier()
        return 0

    lax.fori_loop(0, nsplit, split_body, 0)


def _suffix_small(bufa, bufb, npad, nreal, rank):
    lane = lax.iota(jnp.int32, 16)
    if nreal < npad:
        z = jnp.zeros((16,), jnp.int32)

        def zb(i, _):
            bufa[pl.ds(nreal + i * 16, 16)] = z
            bufb[pl.ds(nreal + i * 16, 16)] = z
            return 0

        lax.fori_loop(0, (npad - nreal) // 16, zb, 0)

    def inner(j, carry):
        run, found, bsel, rrem = carry
        lb = npad - (j + 1) * 16
        c16 = bufa[pl.ds(lb, 16)] + bufb[pl.ds(lb, 16)]
        crev = lax.rev(c16, (0,))
        ssum = plsc.cumsum(crev) + run
        ge = ssum >= rank
        one = jnp.int32(1)
        zero = jnp.int32(0)
        any_ge = jnp.max(jnp.where(ge, one, zero)) > 0
        j_idx = jnp.min(jnp.where(ge, lane, jnp.int32(99)))
        s_at = jnp.max(jnp.where(lane == j_idx, ssum, zero))
        c_at = jnp.max(jnp.where(lane == j_idx, crev, zero))
        hit = jnp.logical_and(any_ge, found == 0)
        bsel = jnp.where(hit, lb + 15 - j_idx, bsel)
        rrem = jnp.where(hit, rank - (s_at - c_at), rrem)
        found = jnp.where(any_ge, one, found)
        run = jnp.max(ssum)
        return run, found, bsel, rrem

    init = (jnp.int32(0), jnp.int32(0), jnp.int32(0), jnp.int32(1))
    _, _, bsel, rrem = lax.fori_loop(0, npad // 16, inner, init)
    return bsel, rrem


def _pass_scratch(npad, nsplit, extra):
    nbw = npad * 16
    return [
        pltpu.VMEM((_CHUNK,), jnp.float32),
        pltpu.VMEM((_CHUNK,), jnp.float32),
        pltpu.VMEM((nbw,), jnp.int32),
        pltpu.VMEM_SHARED((16, nbw // nsplit), jnp.int32),
        pltpu.VMEM((nbw // nsplit // 16,), jnp.int32),
        pltpu.VMEM((nbw // nsplit // 16,), jnp.int32),
        pltpu.VMEM((256,), jnp.int32),
    ] + extra + [
        pltpu.SemaphoreType.DMA,
        pltpu.SemaphoreType.DMA,
    ]


def _sc_pass_a(acts_flat):
    def body(acts_hbm, out_hbm, buf0, buf1, hist, shared, red, tmp, fold,
             sem0, sem1):
        _zero_i32(hist, _PA_PAD * 16)

        def idx_fn(bits, lane):
            return (lax.shift_right_logical(bits, 15)
                    & jnp.int32(0xFFF0)) | lane

        _hist_shard(acts_hbm, hist, buf0, buf1, sem0, sem1, idx_fn)
        _reduce_fold(hist, shared, red, tmp, fold, out_hbm,
                     _PA_PAD, _PA_SPLIT)

    return pl.kernel(
        body,
        out_type=jax.ShapeDtypeStruct((2 * _PA_PAD,), jnp.int32),
        mesh=_sc_mesh(),
        compiler_params=pltpu.CompilerParams(needs_layout_passes=False),
        scratch_types=_pass_scratch(_PA_PAD, _PA_SPLIT, []),
    )(acts_flat)


def _sc_pass_b(acts_flat, ha):
    def body(acts_hbm, ha_hbm, out_hbm, buf0, buf1, hist, shared, red, tmp,
             fold, sba, sbb, sem0, sem1):
        pltpu.sync_copy(ha_hbm.at[pl.ds(0, _PA_PAD)], sba)
        pltpu.sync_copy(ha_hbm.at[pl.ds(_PA_PAD, _PA_PAD)], sbb)
        b1, _r1 = _suffix_small(sba, sbb, _PA_PAD, _PA_PAD,
                                jnp.int32(K_TOTAL))
        b1vec = jnp.full((16,), b1, jnp.int32)
        _zero_i32(hist, _PB_PAD * 16)

        def idx_fn(bits, lane):
            m = bits & jnp.int32(0x7FFFFFFF)
            hi = lax.shift_right_logical(m, 19)
            mid4 = lax.shift_right_logical(m, 4) & jnp.int32(0x7FF0)
            return jnp.where(hi == b1vec, mid4,
                             jnp.int32(_PB_BINS * 16)) | lane

        _hist_shard(acts_hbm, hist, buf0, buf1, sem0, sem1, idx_fn)
        _reduce_fold(hist, shared, red, tmp, fold, out_hbm,
                     _PB_PAD, _PB_SPLIT)

    return pl.kernel(
        body,
        out_type=jax.ShapeDtypeStruct((2 * _PB_PAD,), jnp.int32),
        mesh=_sc_mesh(),
        compiler_params=pltpu.CompilerParams(needs_layout_passes=False),
        scratch_types=_pass_scratch(_PB_PAD, _PB_SPLIT, [
            pltpu.VMEM((_PA_PAD,), jnp.int32),
            pltpu.VMEM((_PA_PAD,), jnp.int32),
        ]),
    )(acts_flat, ha)


def _sc_pass_c(acts_flat, ha, hb):
    def body(acts_hbm, ha_hbm, hb_hbm, out_hbm, sel_hbm, buf0, buf1, hist,
             shared, red, tmp, fold, sba, sbb, outv, sem0, sem1):
        pltpu.sync_copy(ha_hbm.at[pl.ds(0, _PA_PAD)], sba.at[pl.ds(0, _PA_PAD)])
        pltpu.sync_copy(ha_hbm.at[pl.ds(_PA_PAD, _PA_PAD)],
                        sbb.at[pl.ds(0, _PA_PAD)])
        b1, _r1 = _suffix_small(sba, sbb, _PA_PAD, _PA_PAD,
                                jnp.int32(K_TOTAL))
        pltpu.sync_copy(hb_hbm.at[pl.ds(0, _PB_PAD)], sba.at[pl.ds(0, _PB_PAD)])
        pltpu.sync_copy(hb_hbm.at[pl.ds(_PB_PAD, _PB_PAD)],
                        sbb.at[pl.ds(0, _PB_PAD)])
        b2, r2 = _suffix_small(sba, sbb, _PB_PAD, _PB_BINS, _r1)
        bfull = lax.shift_left(b1, 11) | b2
        bvec = jnp.full((16,), bfull, jnp.int32)
        _zero_i32(hist, _PC_PAD * 16)

        def idx_fn(bits, lane):
            m = bits & jnp.int32(0x7FFFFFFF)
            hi23 = lax.shift_right_logical(m, 8)
            low4 = lax.shift_left(m, 4) & jnp.int32(0xFF0)
            return jnp.where(hi23 == bvec, low4,
                             jnp.int32(_PC_BINS * 16)) | lane

        _hist_shard(acts_hbm, hist, buf0, buf1, sem0, sem1, idx_fn)
        _reduce_fold(hist, shared, red, tmp, fold, out_hbm,
                     _PC_PAD, _PC_SPLIT)

        @pl.when(jnp.logical_and(lax.axis_index("c") == 0,
                                 lax.axis_index("s") == 0))
        def _():
            outv[pl.ds(0, 16)] = jnp.full((16,), bfull, jnp.int32)
            outv[pl.ds(16, 16)] = jnp.full((16,), r2, jnp.int32)
            pltpu.sync_copy(outv, sel_hbm)

    return pl.kernel(
        body,
        out_type=[
            jax.ShapeDtypeStruct((2 * _PC_PAD,), jnp.int32),
            jax.ShapeDtypeStruct((32,), jnp.int32),
        ],
        mesh=_sc_mesh(),
        compiler_params=pltpu.CompilerParams(needs_layout_passes=False),
        scratch_types=_pass_scratch(_PC_PAD, _PC_SPLIT, [
            pltpu.VMEM((_PA_PAD,), jnp.int32),
            pltpu.VMEM((_PA_PAD,), jnp.int32),
            pltpu.VMEM((32,), jnp.int32),
        ]),
    )(acts_flat, ha, hb)


def _thr_body(sel_ref, hc_ref, o_ref):
    h = hc_ref[...].astype(jnp.float32)
    hsum = jnp.sum(h, axis=0, keepdims=True)
    lanes = lax.broadcasted_iota(jnp.int32, (1, _PC_PAD), 1)
    hm = jnp.where(lanes < _PC_BINS, hsum, 0.0)
    rows = lax.broadcasted_iota(jnp.int32, (_PC_PAD, _PC_PAD), 0)
    cols = lax.broadcasted_iota(jnp.int32, (_PC_PAD, _PC_PAD), 1)
    tri = jnp.where(rows >= cols, 1.0, 0.0).astype(jnp.float32)
    suffix = lax.dot_general(hm, tri, (((1,), (0,)), ((), ())),
                             preferred_element_type=jnp.float32)
    r2 = sel_ref[1, 0].astype(jnp.float32)
    ok = jnp.logical_and(suffix >= r2, lanes < _PC_BINS)
    b3 = jnp.max(jnp.where(ok, lanes, 0))
    bits = lax.shift_left(sel_ref[0, 0], 8) | b3
    o_ref[...] = jnp.full((1, 1), bits, jnp.int32)


def _thr_bits_kernel(sel, hc):
    return pl.pallas_call(
        _thr_body,
        in_specs=[
            pl.BlockSpec(memory_space=pltpu.SMEM),
            pl.BlockSpec((2, _PC_PAD), lambda: (0, 0)),
        ],
        out_specs=pl.BlockSpec((1, 1), lambda: (0, 0)),
        out_shape=jax.ShapeDtypeStruct((1, 1), jnp.int32),
    )(sel, hc)


def _threshold(acts):
    ha = _sc_pass_a(acts)
    hb = _sc_pass_b(acts, ha)
    hc, sel = _sc_pass_c(acts, ha, hb)
    thr_bits = _thr_bits_kernel(sel.reshape(2, 16), hc.reshape(2, _PC_PAD))
    return lax.bitcast_convert_type(thr_bits, jnp.float32)


def kernel(x, W_enc, W_dec, b_dec):
    acts = _encode(x, W_enc, b_dec)
    thr = _threshold(acts)
    thr2d = thr.reshape(1, 1)
    acts_topk, sae_out, stats = _decode(
        thr2d, acts, W_dec.astype(jnp.bfloat16), x, b_dec)

    s_abs = stats[0, 0]
    s_cnt = stats[0, 1]
    s_pos = stats[0, 2]
    s_neg = stats[0, 3]
    s_l2 = stats[0, 4]

    inv_tok = jnp.float32(1.0 / TOKENS)
    l1_norm = s_abs * inv_tok
    l1_loss = jnp.float32(L1_COEFF) * l1_norm
    l0_norm = s_cnt * inv_tok
    l2_loss = s_l2 * jnp.float32(1.0 / (TOKENS * ACT))
    positive_features = s_pos * inv_tok
    negative_features = s_neg * inv_tok
    loss = l2_loss
    num_dead_features = jnp.zeros((), jnp.int32)
    aux_loss = jnp.zeros((), x.dtype)

    return (sae_out, acts_topk, num_dead_features, loss, l1_loss, l2_loss,
            l0_norm, l1_norm, aux_loss, positive_features, negative_features)


# speedup vs baseline: 1.0325x; 1.0325x over previous
import functools

import jax
import jax.numpy as jnp
from jax import lax
from jax.experimental import pallas as pl
from jax.experimental.pallas import tpu as pltpu
from jax.experimental.pallas import tpu_sc as plsc

ACT = 2048
DICT = 16384
TOKENS = 4096
K = 64
K_TOTAL = K * TOKENS
L1_COEFF = 1e-4
NTOT = TOKENS * DICT


def _encode_body(x_ref, b_ref, w_ref, o_ref):
    xc = x_ref[...] - b_ref[...]
    o_ref[...] = lax.dot_general(
        xc, w_ref[...], (((1,), (0,)), ((), ())),
        preferred_element_type=jnp.float32)


def _encode(x, W_enc, b_dec):
    BM, BN = 2048, 512
    grid = (TOKENS // BM, DICT // BN)
    return pl.pallas_call(
        _encode_body,
        grid=grid,
        in_specs=[
            pl.BlockSpec((BM, ACT), lambda i, j: (i, 0)),
            pl.BlockSpec((1, ACT), lambda i, j: (0, 0)),
            pl.BlockSpec((ACT, BN), lambda i, j: (0, j)),
        ],
        out_specs=pl.BlockSpec((BM, BN), lambda i, j: (i, j)),
        out_shape=jax.ShapeDtypeStruct((TOKENS, DICT), jnp.float32),
    )(x, b_dec.reshape(1, ACT), W_enc)


def _decode_body(thr_ref, a_ref, w_ref, x_ref, b_ref,
                 topk_ref, sae_ref, stats_ref, acc_ref, sacc_ref,
                 *, n_m, n_k):
    m = pl.program_id(0)
    kk = pl.program_id(1)
    t = thr_ref[0, 0]

    a = a_ref[...]
    sel = jnp.abs(a) >= t
    av = jnp.where(sel, a, 0.0)
    topk_ref[...] = av

    @pl.when(jnp.logical_and(m == 0, kk == 0))
    def _init_stats():
        for i in range(5):
            sacc_ref[i] = 0.0

    sacc_ref[0] += jnp.sum(jnp.abs(av))
    sacc_ref[1] += jnp.sum(sel.astype(jnp.float32))
    sacc_ref[2] += jnp.sum((av > 0).astype(jnp.float32))
    sacc_ref[3] += jnp.sum((av < 0).astype(jnp.float32))

    @pl.when(kk == 0)
    def _zero_acc():
        acc_ref[...] = jnp.zeros_like(acc_ref)

    acc_ref[...] += lax.dot_general(
        av.astype(jnp.bfloat16), w_ref[...], (((1,), (0,)), ((), ())),
        preferred_element_type=jnp.float32)

    @pl.when(kk == n_k - 1)
    def _finish_row():
        rec = acc_ref[...] + b_ref[...]
        sae_ref[...] = rec
        sacc_ref[4] += jnp.sum((rec - x_ref[...]) ** 2)

    @pl.when(jnp.logical_and(m == n_m - 1, kk == n_k - 1))
    def _emit_stats():
        lane = lax.broadcasted_iota(jnp.int32, (1, 128), 1)
        out = jnp.zeros((1, 128), jnp.float32)
        for i in range(5):
            out = jnp.where(lane == i, sacc_ref[i], out)
        stats_ref[...] = out


def _decode(thr2d, acts, W_dec_bf16, x, b_dec):
    BM, BK = 512, 1024
    n_m, n_k = TOKENS // BM, DICT // BK
    body = functools.partial(_decode_body, n_m=n_m, n_k=n_k)
    return pl.pallas_call(
        body,
        grid=(n_m, n_k),
        in_specs=[
            pl.BlockSpec(memory_space=pltpu.SMEM),
            pl.BlockSpec((BM, BK), lambda m, k: (m, k)),
            pl.BlockSpec((BK, ACT), lambda m, k: (k, 0)),
            pl.BlockSpec((BM, ACT), lambda m, k: (m, 0)),
            pl.BlockSpec((1, ACT), lambda m, k: (0, 0)),
        ],
        out_specs=[
            pl.BlockSpec((BM, BK), lambda m, k: (m, k)),
            pl.BlockSpec((BM, ACT), lambda m, k: (m, 0)),
            pl.BlockSpec((1, 128), lambda m, k: (0, 0)),
        ],
        out_shape=[
            jax.ShapeDtypeStruct((TOKENS, DICT), jnp.float32),
            jax.ShapeDtypeStruct((TOKENS, ACT), jnp.float32),
            jax.ShapeDtypeStruct((1, 128), jnp.float32),
        ],
        scratch_shapes=[
            pltpu.VMEM((BM, ACT), jnp.float32),
            pltpu.SMEM((8,), jnp.float32),
        ],
    )(thr2d, acts, W_dec_bf16, x, b_dec.reshape(1, ACT))


_NW = 32
_SHARD = NTOT // _NW
_CHUNK = DICT
_NCHUNK = _SHARD // _CHUNK

_PA_BINS, _PA_PAD, _PA_SPLIT = 4096, 4096, 16
_PB_BINS, _PB_PAD, _PB_SPLIT = 2048, 2304, 3
_PC_BINS, _PC_PAD, _PC_SPLIT = 256, 512, 1


def _sc_mesh():
    return plsc.VectorSubcoreMesh(core_axis_name="c", subcore_axis_name="s")


def _worker_id():
    return lax.axis_index("s") * 2 + lax.axis_index("c")


def _zero_i32(ref, n):
    z = jnp.zeros((16,), jnp.int32)

    def bd(i, _):
        for u in range(4):
            ref[pl.ds(i * 64 + u * 16, 16)] = z
        return 0

    lax.fori_loop(0, n // 64, bd, 0)


def _hist_shard(acts_hbm, hist_ref, buf0, buf1, sem0, sem1, idx_fn):
    base = _worker_id() * _NCHUNK
    lane = lax.iota(jnp.int32, 16)
    ones = jnp.ones((16,), jnp.int32)

    def start(buf, sem, row):
        pltpu.async_copy(acts_hbm.at[row], buf, sem)

    def wait(buf, sem, row):
        pltpu.make_async_copy(acts_hbm.at[row], buf, sem).wait()

    def process(buf):
        @plsc.parallel_loop(0, _CHUNK // 16, unroll=8)
        def _body(i):
            v = buf[pl.ds(i * 16, 16)]
            bits = plsc.bitcast(v, jnp.int32)
            idx = idx_fn(bits, lane)
            plsc.addupdate_scatter(hist_ref, [idx], ones)

    start(buf0, sem0, base)

    def outer(i, _):
        row_a = base + 2 * i
        start(buf1, sem1, row_a + 1)
        wait(buf0, sem0, row_a)
        process(buf0)

        @pl.when(i < _NCHUNK // 2 - 1)
        def _():
            start(buf0, sem0, row_a + 2)

        wait(buf1, sem1, row_a + 1)
        process(buf1)
        return 0

    lax.fori_loop(0, _NCHUNK // 2, outer, 0)


def _reduce_fold(hist_ref, shared, red, tmp, fold, out_hbm, npad, nsplit):
    c = lax.axis_index("c")
    s = lax.axis_index("s")
    lane = lax.iota(jnp.int32, 16)
    nbw = npad * 16
    seg = nbw // nsplit
    sl = seg // 16
    bins_pt = sl // 16

    def split_body(split, _):
        pltpu.sync_copy(hist_ref.at[pl.ds(split * seg, seg)], shared.at[s])
        plsc.subcore_barrier()
        pltpu.sync_copy(shared.at[0, pl.ds(s * sl, sl)], red)

        def row_body(row, _):
            pltpu.sync_copy(shared.at[row, pl.ds(s * sl, sl)], tmp)

            def add_bd(i, _):
                red[pl.ds(i * 16, 16)] += tmp[pl.ds(i * 16, 16)]
                return 0

            lax.fori_loop(0, sl // 16, add_bd, 0)
            return 0

        lax.fori_loop(1, 16, row_body, 0)

        def fold_g(g, _):
            def fold_j(j, acc):
                vec = red[pl.ds((g * 16 + j) * 16, 16)]
                return jnp.where(lane == j, jnp.sum(vec), acc)

            acc = lax.fori_loop(0, 16, fold_j, jnp.zeros((16,), jnp.int32))
            fold[pl.ds(g * 16, 16)] = acc
            return 0

        lax.fori_loop(0, bins_pt // 16, fold_g, 0)
        pltpu.sync_copy(
            fold.at[pl.ds(0, bins_pt)],
            out_hbm.at[pl.ds(c * npad + split * (seg // 16) + s * bins_pt,
                             bins_pt)])
        plsc.subcore_barrier()
        return 0

    lax.fori_loop(0, nsplit, split_body, 0)


def _suffix_small(bufa, bufb, npad, nreal, rank):
    lane = lax.iota(jnp.int32, 16)
    if nreal < npad:
        z = jnp.zeros((16,), jnp.int32)

        def zb(i, _):
            bufa[pl.ds(nreal + i * 16, 16)] = z
            bufb[pl.ds(nreal + i * 16, 16)] = z
            return 0

        lax.fori_loop(0, (npad - nreal) // 16, zb, 0)

    def inner(j, carry):
        run, found, bsel, rrem = carry
        lb = npad - (j + 1) * 16
        c16 = bufa[pl.ds(lb, 16)] + bufb[pl.ds(lb, 16)]
        crev = lax.rev(c16, (0,))
        ssum = plsc.cumsum(crev) + run
        ge = ssum >= rank
        one = jnp.int32(1)
        zero = jnp.int32(0)
        any_ge = jnp.max(jnp.where(ge, one, zero)) > 0
        j_idx = jnp.min(jnp.where(ge, lane, jnp.int32(99)))
        s_at = jnp.max(jnp.where(lane == j_idx, ssum, zero))
        c_at = jnp.max(jnp.where(lane == j_idx, crev, zero))
        hit = jnp.logical_and(any_ge, found == 0)
        bsel = jnp.where(hit, lb + 15 - j_idx, bsel)
        rrem = jnp.where(hit, rank - (s_at - c_at), rrem)
        found = jnp.where(any_ge, one, found)
        run = jnp.max(ssum)
        return run, found, bsel, rrem

    init = (jnp.int32(0), jnp.int32(0), jnp.int32(0), jnp.int32(1))
    _, _, bsel, rrem = lax.fori_loop(0, npad // 16, inner, init)
    return bsel, rrem


def _pass_scratch(npad, nsplit, extra):
    nbw = npad * 16
    return [
        pltpu.VMEM((_CHUNK,), jnp.float32),
        pltpu.VMEM((_CHUNK,), jnp.float32),
        pltpu.VMEM((nbw,), jnp.int32),
        pltpu.VMEM_SHARED((16, nbw // nsplit), jnp.int32),
        pltpu.VMEM((nbw // nsplit // 16,), jnp.int32),
        pltpu.VMEM((nbw // nsplit // 16,), jnp.int32),
        pltpu.VMEM((256,), jnp.int32),
    ] + extra + [
        pltpu.SemaphoreType.DMA,
        pltpu.SemaphoreType.DMA,
    ]


def _sc_pass_a(acts_flat):
    def body(acts_hbm, out_hbm, buf0, buf1, hist, shared, red, tmp, fold,
             sem0, sem1):
        _zero_i32(hist, _PA_PAD * 16)

        def idx_fn(bits, lane):
            return (lax.shift_right_logical(bits, 15)
                    & jnp.int32(0xFFF0)) | lane

        _hist_shard(acts_hbm, hist, buf0, buf1, sem0, sem1, idx_fn)
        _reduce_fold(hist, shared, red, tmp, fold, out_hbm,
                     _PA_PAD, _PA_SPLIT)

    return pl.kernel(
        body,
        out_type=jax.ShapeDtypeStruct((2 * _PA_PAD,), jnp.int32),
        mesh=_sc_mesh(),
        compiler_params=pltpu.CompilerParams(needs_layout_passes=False),
        scratch_types=_pass_scratch(_PA_PAD, _PA_SPLIT, []),
    )(acts_flat)


def _sc_pass_b(acts_flat, ha):
    def body(acts_hbm, ha_hbm, out_hbm, buf0, buf1, hist, shared, red, tmp,
             fold, sba, sbb, sem0, sem1):
        pltpu.sync_copy(ha_hbm.at[pl.ds(0, _PA_PAD)], sba)
        pltpu.sync_copy(ha_hbm.at[pl.ds(_PA_PAD, _PA_PAD)], sbb)
        b1, _r1 = _suffix_small(sba, sbb, _PA_PAD, _PA_PAD,
                                jnp.int32(K_TOTAL))
        b1vec = jnp.full((16,), b1, jnp.int32)
        _zero_i32(hist, _PB_PAD * 16)

        def idx_fn(bits, lane):
            m = bits & jnp.int32(0x7FFFFFFF)
            hi = lax.shift_right_logical(m, 19)
            mid4 = lax.shift_right_logical(m, 4) & jnp.int32(0x7FF0)
            return jnp.where(hi == b1vec, mid4,
                             jnp.int32(_PB_BINS * 16)) | lane

        _hist_shard(acts_hbm, hist, buf0, buf1, sem0, sem1, idx_fn)
        _reduce_fold(hist, shared, red, tmp, fold, out_hbm,
                     _PB_PAD, _PB_SPLIT)

    return pl.kernel(
        body,
        out_type=jax.ShapeDtypeStruct((2 * _PB_PAD,), jnp.int32),
        mesh=_sc_mesh(),
        compiler_params=pltpu.CompilerParams(needs_layout_passes=False),
        scratch_types=_pass_scratch(_PB_PAD, _PB_SPLIT, [
            pltpu.VMEM((_PA_PAD,), jnp.int32),
            pltpu.VMEM((_PA_PAD,), jnp.int32),
        ]),
    )(acts_flat, ha)


def _sc_pass_c(acts_flat, ha, hb):
    def body(acts_hbm, ha_hbm, hb_hbm, out_hbm, sel_hbm, buf0, buf1, hist,
             shared, red, tmp, fold, sba, sbb, outv, sem0, sem1):
        pltpu.sync_copy(ha_hbm.at[pl.ds(0, _PA_PAD)], sba.at[pl.ds(0, _PA_PAD)])
        pltpu.sync_copy(ha_hbm.at[pl.ds(_PA_PAD, _PA_PAD)],
                        sbb.at[pl.ds(0, _PA_PAD)])
        b1, _r1 = _suffix_small(sba, sbb, _PA_PAD, _PA_PAD,
                                jnp.int32(K_TOTAL))
        pltpu.sync_copy(hb_hbm.at[pl.ds(0, _PB_PAD)], sba.at[pl.ds(0, _PB_PAD)])
        pltpu.sync_copy(hb_hbm.at[pl.ds(_PB_PAD, _PB_PAD)],
                        sbb.at[pl.ds(0, _PB_PAD)])
        b2, r2 = _suffix_small(sba, sbb, _PB_PAD, _PB_BINS, _r1)
        bfull = lax.shift_left(b1, 11) | b2
        bvec = jnp.full((16,), bfull, jnp.int32)
        _zero_i32(hist, _PC_PAD * 16)

        def idx_fn(bits, lane):
            m = bits & jnp.int32(0x7FFFFFFF)
            hi23 = lax.shift_right_logical(m, 8)
            low4 = lax.shift_left(m, 4) & jnp.int32(0xFF0)
            return jnp.where(hi23 == bvec, low4,
                             jnp.int32(_PC_BINS * 16)) | lane

        _hist_shard(acts_hbm, hist, buf0, buf1, sem0, sem1, idx_fn)
        _reduce_fold(hist, shared, red, tmp, fold, out_hbm,
                     _PC_PAD, _PC_SPLIT)

        @pl.when(jnp.logical_and(lax.axis_index("c") == 0,
                                 lax.axis_index("s") == 0))
        def _():
            outv[pl.ds(0, 16)] = jnp.full((16,), bfull, jnp.int32)
            outv[pl.ds(16, 16)] = jnp.full((16,), r2, jnp.int32)
            pltpu.sync_copy(outv, sel_hbm)

    return pl.kernel(
        body,
        out_type=[
            jax.ShapeDtypeStruct((2 * _PC_PAD,), jnp.int32),
            jax.ShapeDtypeStruct((32,), jnp.int32),
        ],
        mesh=_sc_mesh(),
        compiler_params=pltpu.CompilerParams(needs_layout_passes=False),
        scratch_types=_pass_scratch(_PC_PAD, _PC_SPLIT, [
            pltpu.VMEM((_PA_PAD,), jnp.int32),
            pltpu.VMEM((_PA_PAD,), jnp.int32),
            pltpu.VMEM((32,), jnp.int32),
        ]),
    )(acts_flat, ha, hb)


def _thr_body(sel_ref, hc_ref, o_ref):
    h = hc_ref[...].astype(jnp.float32)
    hsum = jnp.sum(h, axis=0, keepdims=True)
    lanes = lax.broadcasted_iota(jnp.int32, (1, _PC_PAD), 1)
    hm = jnp.where(lanes < _PC_BINS, hsum, 0.0)
    rows = lax.broadcasted_iota(jnp.int32, (_PC_PAD, _PC_PAD), 0)
    cols = lax.broadcasted_iota(jnp.int32, (_PC_PAD, _PC_PAD), 1)
    tri = jnp.where(rows >= cols, 1.0, 0.0).astype(jnp.float32)
    suffix = lax.dot_general(hm, tri, (((1,), (0,)), ((), ())),
                             preferred_element_type=jnp.float32)
    r2 = sel_ref[1, 0].astype(jnp.float32)
    ok = jnp.logical_and(suffix >= r2, lanes < _PC_BINS)
    b3 = jnp.max(jnp.where(ok, lanes, 0))
    bits = lax.shift_left(sel_ref[0, 0], 8) | b3
    o_ref[...] = jnp.full((1, 1), bits, jnp.int32)


def _thr_bits_kernel(sel, hc):
    return pl.pallas_call(
        _thr_body,
        in_specs=[
            pl.BlockSpec(memory_space=pltpu.SMEM),
            pl.BlockSpec((2, _PC_PAD), lambda: (0, 0)),
        ],
        out_specs=pl.BlockSpec((1, 1), lambda: (0, 0)),
        out_shape=jax.ShapeDtypeStruct((1, 1), jnp.int32),
    )(sel, hc)


def _threshold(acts):
    ha = _sc_pass_a(acts)
    hb = _sc_pass_b(acts, ha)
    hc, sel = _sc_pass_c(acts, ha, hb)
    thr_bits = _thr_bits_kernel(sel.reshape(2, 16), hc.reshape(2, _PC_PAD))
    return lax.bitcast_convert_type(thr_bits, jnp.float32)


def kernel(x, W_enc, W_dec, b_dec):
    acts = _encode(x, W_enc, b_dec)
    thr = _threshold(acts)
    thr2d = thr.reshape(1, 1)
    acts_topk, sae_out, stats = _decode(
        thr2d, acts, W_dec.astype(jnp.bfloat16), x, b_dec)

    s_abs = stats[0, 0]
    s_cnt = stats[0, 1]
    s_pos = stats[0, 2]
    s_neg = stats[0, 3]
    s_l2 = stats[0, 4]

    inv_tok = jnp.float32(1.0 / TOKENS)
    l1_norm = s_abs * inv_tok
    l1_loss = jnp.float32(L1_COEFF) * l1_norm
    l0_norm = s_cnt * inv_tok
    l2_loss = s_l2 * jnp.float32(1.0 / (TOKENS * ACT))
    positive_features = s_pos * inv_tok
    negative_features = s_neg * inv_tok
    loss = l2_loss
    num_dead_features = jnp.zeros((), jnp.int32)
    aux_loss = jnp.zeros((), x.dtype)

    return (sae_out, acts_topk, num_dead_features, loss, l1_loss, l2_loss,
            l0_norm, l1_norm, aux_loss, positive_features, negative_features)
